# baseline (speedup 1.0000x reference)
import functools

import jax
import jax.numpy as jnp
from jax import lax
from jax.experimental import pallas as pl
from jax.experimental.pallas import tpu as pltpu
from jax.experimental.pallas import tpu_sc as plsc

N = 10000
E = 160000
IN = 128
OUT = 8
HID = 800

TILES = 32
EC = 64
CHUNKS_PER_TILE = 80
EPAD = TILES * CHUNKS_PER_TILE * EC
NROWS = 10240
ROWS_PER_TILE = NROWS // 16
RCHUNKS = ROWS_PER_TILE // EC
D2 = 16


def _sc_kernel_pass1():
  mesh = plsc.VectorSubcoreMesh(core_axis_name="c", subcore_axis_name="s")

  @functools.partial(
      pl.kernel,
      out_type=(
          jax.ShapeDtypeStruct((2, NROWS, IN), jnp.float32),
          jax.ShapeDtypeStruct((2, NROWS, D2), jnp.float32),
      ),
      mesh=mesh,
      compiler_params=pltpu.CompilerParams(use_tc_tiling_on_sc=False),
      scratch_types=[
          pltpu.VMEM((CHUNKS_PER_TILE, EC), jnp.int32),
          pltpu.VMEM((CHUNKS_PER_TILE, EC), jnp.int32),
          pltpu.VMEM((EC, IN), jnp.float32),
          pltpu.VMEM((EC, IN), jnp.float32),
          pltpu.VMEM((EC, D2), jnp.float32),
          pltpu.VMEM((EC, D2), jnp.float32),
          pltpu.VMEM_SHARED((NROWS, IN), jnp.float32),
          pltpu.VMEM_SHARED((NROWS, D2), jnp.float32),
          pltpu.SemaphoreType.DMA,
          pltpu.SemaphoreType.DMA,
      ],
  )
  def seg(vals_hbm, src_hbm, dst_hbm, zeros_hbm, zd_hbm, ones_hbm,
          out_hbm, deg_hbm,
          src_v, dst_v, rows_a, rows_b, ones_v, zbuf, acc, acc_deg,
          sem_a, sem_b):
    cid = lax.axis_index("c")
    sid = lax.axis_index("s")
    wid = sid * 2 + cid
    base = sid * ROWS_PER_TILE
    with jax.named_scope("p1_zero"):
      pltpu.sync_copy(zeros_hbm, rows_a)
      pltpu.sync_copy(zd_hbm, zbuf)
      pltpu.sync_copy(ones_hbm, ones_v)

      def zbody(j, carry):
        pltpu.sync_copy(rows_a, acc.at[pl.ds(base + j * EC, EC)])
        pltpu.sync_copy(zbuf, acc_deg.at[pl.ds(base + j * EC, EC)])
        return carry

      lax.fori_loop(0, RCHUNKS, zbody, 0)
    with jax.named_scope("p1_idx"):
      pltpu.sync_copy(
          src_hbm.at[pl.ds(wid * CHUNKS_PER_TILE, CHUNKS_PER_TILE)], src_v)
      pltpu.sync_copy(
          dst_hbm.at[pl.ds(wid * CHUNKS_PER_TILE, CHUNKS_PER_TILE)], dst_v)
      plsc.subcore_barrier()

    def body(jj, carry):
      j0 = 2 * jj
      j1 = 2 * jj + 1
      ga = pltpu.async_copy(vals_hbm.at[src_v.at[j0]], rows_a, sem_a)
      gb = pltpu.async_copy(vals_hbm.at[src_v.at[j1]], rows_b, sem_b)
      pltpu.sync_copy(ones_v, acc_deg.at[dst_v.at[j0]], add=True)
      ga.wait()
      pltpu.sync_copy(rows_a, acc.at[dst_v.at[j0]], add=True)
      pltpu.sync_copy(ones_v, acc_deg.at[dst_v.at[j1]], add=True)
      gb.wait()
      pltpu.sync_copy(rows_b, acc.at[dst_v.at[j1]], add=True)
      return carry

    with jax.named_scope("p1_edges"):
      lax.fori_loop(0, CHUNKS_PER_TILE // 2, body, 0)
      plsc.subcore_barrier()

    def rbody(j, carry):
      pltpu.sync_copy(acc.at[pl.ds(base + j * EC, EC)], rows_a)
      pltpu.sync_copy(rows_a, out_hbm.at[cid, pl.ds(base + j * EC, EC)])
      pltpu.sync_copy(acc_deg.at[pl.ds(base + j * EC, EC)], zbuf)
      pltpu.sync_copy(zbuf, deg_hbm.at[cid, pl.ds(base + j * EC, EC)])
      return carry

    with jax.named_scope("p1_read"):
      lax.fori_loop(0, RCHUNKS, rbody, 0)

  return seg


def _sc_kernel_pass2():
  mesh = plsc.VectorSubcoreMesh(core_axis_name="c", subcore_axis_name="s")

  @functools.partial(
      pl.kernel,
      out_type=jax.ShapeDtypeStruct((2, NROWS, D2), jnp.float32),
      mesh=mesh,
      compiler_params=pltpu.CompilerParams(use_tc_tiling_on_sc=False),
      scratch_types=[
          pltpu.VMEM((CHUNKS_PER_TILE, EC), jnp.int32),
          pltpu.VMEM((CHUNKS_PER_TILE, EC), jnp.int32),
          pltpu.VMEM((EC, D2), jnp.float32),
          pltpu.VMEM((EC, D2), jnp.float32),
          pltpu.VMEM_SHARED((NROWS, D2), jnp.float32),
          pltpu.SemaphoreType.DMA,
          pltpu.SemaphoreType.DMA,
      ],
  )
  def seg(vals_hbm, src_hbm, dst_hbm, zd_hbm, out_hbm,
          src_v, dst_v, rows_a, rows_b, acc, sem_a, sem_b):
    cid = lax.axis_index("c")
    sid = lax.axis_index("s")
    wid = sid * 2 + cid
    base = sid * ROWS_PER_TILE
    pltpu.sync_copy(zd_hbm, rows_a)

    def zbody(j, carry):
      pltpu.sync_copy(rows_a, acc.at[pl.ds(base + j * EC, EC)])
      return carry

    lax.fori_loop(0, RCHUNKS, zbody, 0)
    pltpu.sync_copy(src_hbm.at[pl.ds(wid * CHUNKS_PER_TILE, CHUNKS_PER_TILE)],
                    src_v)
    pltpu.sync_copy(dst_hbm.at[pl.ds(wid * CHUNKS_PER_TILE, CHUNKS_PER_TILE)],
                    dst_v)
    plsc.subcore_barrier()

    def body(jj, carry):
      j0 = 2 * jj
      j1 = 2 * jj + 1
      ga = pltpu.async_copy(vals_hbm.at[src_v.at[j0]], rows_a, sem_a)
      gb = pltpu.async_copy(vals_hbm.at[src_v.at[j1]], rows_b, sem_b)
      ga.wait()
      pltpu.sync_copy(rows_a, acc.at[dst_v.at[j0]], add=True)
      gb.wait()
      pltpu.sync_copy(rows_b, acc.at[dst_v.at[j1]], add=True)
      return carry

    lax.fori_loop(0, CHUNKS_PER_TILE // 2, body, 0)
    plsc.subcore_barrier()

    def rbody(j, carry):
      pltpu.sync_copy(acc.at[pl.ds(base + j * EC, EC)], rows_a)
      pltpu.sync_copy(rows_a, out_hbm.at[cid, pl.ds(base + j * EC, EC)])
      return carry

    lax.fori_loop(0, RCHUNKS, rbody, 0)

  return seg


_sc_cache = {}


def _sc_pass1(*args):
  if 1 not in _sc_cache:
    _sc_cache[1] = _sc_kernel_pass1()
  return _sc_cache[1](*args)


def _sc_pass2(*args):
  if 2 not in _sc_cache:
    _sc_cache[2] = _sc_kernel_pass2()
  return _sc_cache[2](*args)


_BLK = 1024


def _dense1_body(part, degp, x, w1, w2, w3, b1, b3, wc2, b12, b32,
                 p_out, r_out):
  gx = part[0] + part[1]
  deg = degp[0, :, 0:1] + degp[1, :, 0:1]
  h = jnp.dot(gx, w1[:], preferred_element_type=jnp.float32)
  h = h + jnp.dot(x[:] * (-deg), w2[:], preferred_element_type=jnp.float32)
  h = h + jnp.dot(x[:], w3[:], preferred_element_type=jnp.float32)
  h = h + deg * b1[:] + b3[:]
  h = jnp.maximum(h, 0.0)
  m2 = jnp.dot(h, wc2[:], preferred_element_type=jnp.float32)
  p = m2[:, 0:OUT]
  p_out[:] = jnp.concatenate([p, jnp.zeros_like(p)], axis=1)
  r_out[:] = deg * b12[:] - deg * m2[:, OUT:2 * OUT] + m2[:, 2 * OUT:3 * OUT] \
      + b32[:]


def _dense2_body(gp, r, o):
  s = gp[0, :, 0:OUT] + gp[1, :, 0:OUT] + r[:]
  o[:] = jnp.maximum(s, 0.0)


def kernel(x, edge_index, W1_1, b1_1, W2_1, W3_1, b3_1,
           W1_2, b1_2, W2_2, W3_2, b3_2):
  src = edge_index[0]
  dst = edge_index[1]
  pad = EPAD - E
  src_p = jnp.concatenate([src, jnp.zeros((pad,), jnp.int32)])
  dst_p = jnp.concatenate([dst, jnp.full((pad,), NROWS - 1, jnp.int32)])
  src_p = src_p.reshape(EPAD // EC, EC)
  dst_p = dst_p.reshape(EPAD // EC, EC)

  zeros1 = jnp.zeros((EC, IN), jnp.float32)
  zerosd = jnp.zeros((EC, D2), jnp.float32)
  ones1 = jnp.ones((EC, D2), jnp.float32)
  part1, degp = _sc_pass1(x, src_p, dst_p, zeros1, zerosd, ones1)

  wc2 = jnp.concatenate([W1_2, W2_2, W3_2], axis=1)
  grid = NROWS // _BLK
  p_pad, r = pl.pallas_call(
      _dense1_body,
      grid=(grid,),
      in_specs=[
          pl.BlockSpec((2, _BLK, IN), lambda i: (0, i, 0)),
          pl.BlockSpec((2, _BLK, D2), lambda i: (0, i, 0)),
          pl.BlockSpec((_BLK, IN), lambda i: (i, 0)),
          pl.BlockSpec((IN, HID), lambda i: (0, 0)),
          pl.BlockSpec((IN, HID), lambda i: (0, 0)),
          pl.BlockSpec((IN, HID), lambda i: (0, 0)),
          pl.BlockSpec((1, HID), lambda i: (0, 0)),
          pl.BlockSpec((1, HID), lambda i: (0, 0)),
          pl.BlockSpec((HID, 3 * OUT), lambda i: (0, 0)),
          pl.BlockSpec((1, OUT), lambda i: (0, 0)),
          pl.BlockSpec((1, OUT), lambda i: (0, 0)),
      ],
      out_specs=[
          pl.BlockSpec((_BLK, D2), lambda i: (i, 0)),
          pl.BlockSpec((_BLK, OUT), lambda i: (i, 0)),
      ],
      out_shape=[
          jax.ShapeDtypeStruct((NROWS, D2), jnp.float32),
          jax.ShapeDtypeStruct((NROWS, OUT), jnp.float32),
      ],
  )(part1, degp, _pad_rows(x), W1_1, W2_1, W3_1, b1_1.reshape(1, HID),
    b3_1.reshape(1, HID), wc2, b1_2.reshape(1, OUT), b3_2.reshape(1, OUT))

  part2 = _sc_pass2(p_pad, src_p, dst_p, zerosd)

  out = pl.pallas_call(
      _dense2_body,
      grid=(grid,),
      in_specs=[
          pl.BlockSpec((2, _BLK, D2), lambda i: (0, i, 0)),
          pl.BlockSpec((_BLK, OUT), lambda i: (i, 0)),
      ],
      out_specs=pl.BlockSpec((_BLK, OUT), lambda i: (i, 0)),
      out_shape=jax.ShapeDtypeStruct((NROWS, OUT), jnp.float32),
  )(part2, r)
  return out[:N]


def _pad_rows(x):
  return jnp.concatenate(
      [x, jnp.zeros((NROWS - N, x.shape[1]), x.dtype)], axis=0)


# speedup vs baseline: 1.3171x; 1.3171x over previous
import functools

import jax
import jax.numpy as jnp
from jax import lax
from jax.experimental import pallas as pl
from jax.experimental.pallas import tpu as pltpu
from jax.experimental.pallas import tpu_sc as plsc

N = 10000
E = 160000
IN = 128
OUT = 8
HID = 800

TILES = 32
EC = 64
CHUNKS_PER_TILE = 80
EPAD = TILES * CHUNKS_PER_TILE * EC
NROWS = 10240
ROWS_PER_TILE = NROWS // 16
RCHUNKS = ROWS_PER_TILE // EC
D2 = 16


def _sc_kernel_pass1():
  mesh = plsc.VectorSubcoreMesh(core_axis_name="c", subcore_axis_name="s")

  @functools.partial(
      pl.kernel,
      out_type=(
          jax.ShapeDtypeStruct((2, NROWS, IN), jnp.bfloat16),
          jax.ShapeDtypeStruct((2, NROWS, D2), jnp.float32),
      ),
      mesh=mesh,
      compiler_params=pltpu.CompilerParams(use_tc_tiling_on_sc=False),
      scratch_types=[
          pltpu.VMEM((CHUNKS_PER_TILE, EC), jnp.int32),
          pltpu.VMEM((CHUNKS_PER_TILE, EC), jnp.int32),
          pltpu.VMEM((EC, IN), jnp.bfloat16),
          pltpu.VMEM((EC, IN), jnp.bfloat16),
          pltpu.VMEM((EC, D2), jnp.float32),
          pltpu.VMEM((EC, D2), jnp.float32),
          pltpu.VMEM_SHARED((NROWS, IN), jnp.bfloat16),
          pltpu.VMEM_SHARED((NROWS, D2), jnp.float32),
          pltpu.SemaphoreType.DMA,
          pltpu.SemaphoreType.DMA,
      ],
  )
  def seg(vals_hbm, src_hbm, dst_hbm, zeros_hbm, zd_hbm, ones_hbm,
          out_hbm, deg_hbm,
          src_v, dst_v, rows_a, rows_b, ones_v, zbuf, acc, acc_deg,
          sem_a, sem_b):
    cid = lax.axis_index("c")
    sid = lax.axis_index("s")
    wid = sid * 2 + cid
    base = sid * ROWS_PER_TILE
    with jax.named_scope("p1_zero"):
      pltpu.sync_copy(zeros_hbm, rows_a)
      pltpu.sync_copy(zd_hbm, zbuf)
      pltpu.sync_copy(ones_hbm, ones_v)

      def zbody(j, carry):
        pltpu.sync_copy(rows_a, acc.at[pl.ds(base + j * EC, EC)])
        pltpu.sync_copy(zbuf, acc_deg.at[pl.ds(base + j * EC, EC)])
        return carry

      lax.fori_loop(0, RCHUNKS, zbody, 0)
    with jax.named_scope("p1_idx"):
      pltpu.sync_copy(
          src_hbm.at[pl.ds(wid * CHUNKS_PER_TILE, CHUNKS_PER_TILE)], src_v)
      pltpu.sync_copy(
          dst_hbm.at[pl.ds(wid * CHUNKS_PER_TILE, CHUNKS_PER_TILE)], dst_v)
      plsc.subcore_barrier()

    def body(jj, carry):
      j0 = 2 * jj
      j1 = 2 * jj + 1
      ga = pltpu.async_copy(vals_hbm.at[src_v.at[j0]], rows_a, sem_a)
      gb = pltpu.async_copy(vals_hbm.at[src_v.at[j1]], rows_b, sem_b)
      pltpu.sync_copy(ones_v, acc_deg.at[dst_v.at[j0]], add=True)
      ga.wait()
      pltpu.sync_copy(rows_a, acc.at[dst_v.at[j0]], add=True)
      pltpu.sync_copy(ones_v, acc_deg.at[dst_v.at[j1]], add=True)
      gb.wait()
      pltpu.sync_copy(rows_b, acc.at[dst_v.at[j1]], add=True)
      return carry

    with jax.named_scope("p1_edges"):
      lax.fori_loop(0, CHUNKS_PER_TILE // 2, body, 0)
      plsc.subcore_barrier()

    def rbody(j, carry):
      pltpu.sync_copy(acc.at[pl.ds(base + j * EC, EC)], rows_a)
      pltpu.sync_copy(rows_a, out_hbm.at[cid, pl.ds(base + j * EC, EC)])
      pltpu.sync_copy(acc_deg.at[pl.ds(base + j * EC, EC)], zbuf)
      pltpu.sync_copy(zbuf, deg_hbm.at[cid, pl.ds(base + j * EC, EC)])
      return carry

    with jax.named_scope("p1_read"):
      lax.fori_loop(0, RCHUNKS, rbody, 0)

  return seg


def _sc_kernel_pass2():
  mesh = plsc.VectorSubcoreMesh(core_axis_name="c", subcore_axis_name="s")

  @functools.partial(
      pl.kernel,
      out_type=jax.ShapeDtypeStruct((2, NROWS, D2), jnp.float32),
      mesh=mesh,
      compiler_params=pltpu.CompilerParams(use_tc_tiling_on_sc=False),
      scratch_types=[
          pltpu.VMEM((CHUNKS_PER_TILE, EC), jnp.int32),
          pltpu.VMEM((CHUNKS_PER_TILE, EC), jnp.int32),
          pltpu.VMEM((EC, D2), jnp.float32),
          pltpu.VMEM((EC, D2), jnp.float32),
          pltpu.VMEM_SHARED((NROWS, D2), jnp.float32),
          pltpu.SemaphoreType.DMA,
          pltpu.SemaphoreType.DMA,
      ],
  )
  def seg(vals_hbm, src_hbm, dst_hbm, zd_hbm, out_hbm,
          src_v, dst_v, rows_a, rows_b, acc, sem_a, sem_b):
    cid = lax.axis_index("c")
    sid = lax.axis_index("s")
    wid = sid * 2 + cid
    base = sid * ROWS_PER_TILE
    pltpu.sync_copy(zd_hbm, rows_a)

    def zbody(j, carry):
      pltpu.sync_copy(rows_a, acc.at[pl.ds(base + j * EC, EC)])
      return carry

    lax.fori_loop(0, RCHUNKS, zbody, 0)
    pltpu.sync_copy(src_hbm.at[pl.ds(wid * CHUNKS_PER_TILE, CHUNKS_PER_TILE)],
                    src_v)
    pltpu.sync_copy(dst_hbm.at[pl.ds(wid * CHUNKS_PER_TILE, CHUNKS_PER_TILE)],
                    dst_v)
    plsc.subcore_barrier()

    def body(jj, carry):
      j0 = 2 * jj
      j1 = 2 * jj + 1
      ga = pltpu.async_copy(vals_hbm.at[src_v.at[j0]], rows_a, sem_a)
      gb = pltpu.async_copy(vals_hbm.at[src_v.at[j1]], rows_b, sem_b)
      ga.wait()
      pltpu.sync_copy(rows_a, acc.at[dst_v.at[j0]], add=True)
      gb.wait()
      pltpu.sync_copy(rows_b, acc.at[dst_v.at[j1]], add=True)
      return carry

    lax.fori_loop(0, CHUNKS_PER_TILE // 2, body, 0)
    plsc.subcore_barrier()

    def rbody(j, carry):
      pltpu.sync_copy(acc.at[pl.ds(base + j * EC, EC)], rows_a)
      pltpu.sync_copy(rows_a, out_hbm.at[cid, pl.ds(base + j * EC, EC)])
      return carry

    lax.fori_loop(0, RCHUNKS, rbody, 0)

  return seg


_sc_cache = {}


def _sc_pass1(*args):
  if 1 not in _sc_cache:
    _sc_cache[1] = _sc_kernel_pass1()
  return _sc_cache[1](*args)


def _sc_pass2(*args):
  if 2 not in _sc_cache:
    _sc_cache[2] = _sc_kernel_pass2()
  return _sc_cache[2](*args)


_BLK = 1024


def _dense1_body(part, degp, x, w1, w2, w3, b1, b3, wc2, b12, b32,
                 p_out, r_out):
  gx = part[0].astype(jnp.float32) + part[1].astype(jnp.float32)
  deg = degp[0, :, 0:1] + degp[1, :, 0:1]
  h = jnp.dot(gx, w1[:], preferred_element_type=jnp.float32)
  h = h + jnp.dot(x[:] * (-deg), w2[:], preferred_element_type=jnp.float32)
  h = h + jnp.dot(x[:], w3[:], preferred_element_type=jnp.float32)
  h = h + deg * b1[:] + b3[:]
  h = jnp.maximum(h, 0.0)
  m2 = jnp.dot(h, wc2[:], preferred_element_type=jnp.float32)
  p = m2[:, 0:OUT]
  p_out[:] = jnp.concatenate([p, jnp.zeros_like(p)], axis=1)
  r_out[:] = deg * b12[:] - deg * m2[:, OUT:2 * OUT] + m2[:, 2 * OUT:3 * OUT] \
      + b32[:]


def _dense2_body(gp, r, o):
  s = gp[0, :, 0:OUT] + gp[1, :, 0:OUT] + r[:]
  o[:] = jnp.maximum(s, 0.0)


def kernel(x, edge_index, W1_1, b1_1, W2_1, W3_1, b3_1,
           W1_2, b1_2, W2_2, W3_2, b3_2):
  src = edge_index[0]
  dst = edge_index[1]
  pad = EPAD - E
  src_p = jnp.concatenate([src, jnp.zeros((pad,), jnp.int32)])
  dst_p = jnp.concatenate([dst, jnp.full((pad,), NROWS - 1, jnp.int32)])
  src_p = src_p.reshape(EPAD // EC, EC)
  dst_p = dst_p.reshape(EPAD // EC, EC)

  zeros1 = jnp.zeros((EC, IN), jnp.bfloat16)
  zerosd = jnp.zeros((EC, D2), jnp.float32)
  ones1 = jnp.ones((EC, D2), jnp.float32)
  part1, degp = _sc_pass1(x.astype(jnp.bfloat16), src_p, dst_p,
                          zeros1, zerosd, ones1)

  wc2 = jnp.concatenate([W1_2, W2_2, W3_2], axis=1)
  grid = NROWS // _BLK
  p_pad, r = pl.pallas_call(
      _dense1_body,
      grid=(grid,),
      in_specs=[
          pl.BlockSpec((2, _BLK, IN), lambda i: (0, i, 0)),
          pl.BlockSpec((2, _BLK, D2), lambda i: (0, i, 0)),
          pl.BlockSpec((_BLK, IN), lambda i: (i, 0)),
          pl.BlockSpec((IN, HID), lambda i: (0, 0)),
          pl.BlockSpec((IN, HID), lambda i: (0, 0)),
          pl.BlockSpec((IN, HID), lambda i: (0, 0)),
          pl.BlockSpec((1, HID), lambda i: (0, 0)),
          pl.BlockSpec((1, HID), lambda i: (0, 0)),
          pl.BlockSpec((HID, 3 * OUT), lambda i: (0, 0)),
          pl.BlockSpec((1, OUT), lambda i: (0, 0)),
          pl.BlockSpec((1, OUT), lambda i: (0, 0)),
      ],
      out_specs=[
          pl.BlockSpec((_BLK, D2), lambda i: (i, 0)),
          pl.BlockSpec((_BLK, OUT), lambda i: (i, 0)),
      ],
      out_shape=[
          jax.ShapeDtypeStruct((NROWS, D2), jnp.float32),
          jax.ShapeDtypeStruct((NROWS, OUT), jnp.float32),
      ],
  )(part1, degp, _pad_rows(x), W1_1, W2_1, W3_1, b1_1.reshape(1, HID),
    b3_1.reshape(1, HID), wc2, b1_2.reshape(1, OUT), b3_2.reshape(1, OUT))

  part2 = _sc_pass2(p_pad, src_p, dst_p, zerosd)

  out = pl.pallas_call(
      _dense2_body,
      grid=(grid,),
      in_specs=[
          pl.BlockSpec((2, _BLK, D2), lambda i: (0, i, 0)),
          pl.BlockSpec((_BLK, OUT), lambda i: (i, 0)),
      ],
      out_specs=pl.BlockSpec((_BLK, OUT), lambda i: (i, 0)),
      out_shape=jax.ShapeDtypeStruct((NROWS, OUT), jnp.float32),
  )(part2, r)
  return out[:N]


def _pad_rows(x):
  return jnp.concatenate(
      [x, jnp.zeros((NROWS - N, x.shape[1]), x.dtype)], axis=0)


# speedup vs baseline: 1.4054x; 1.0670x over previous
import functools

import jax
import jax.numpy as jnp
from jax import lax
from jax.experimental import pallas as pl
from jax.experimental.pallas import tpu as pltpu
from jax.experimental.pallas import tpu_sc as plsc

N = 10000
E = 160000
IN = 128
OUT = 8
HID = 800

TILES = 32
EC = 64
CF = 108
CS = 52
EPAD = 16 * (CF + CS) * EC
NROWS = 10240
ROWS_PER_TILE = NROWS // 16
RCHUNKS = ROWS_PER_TILE // EC
D2 = 16


def _sc_kernel_pass1():
  mesh = plsc.VectorSubcoreMesh(core_axis_name="c", subcore_axis_name="s")

  @functools.partial(
      pl.kernel,
      out_type=(
          jax.ShapeDtypeStruct((2, NROWS, IN), jnp.bfloat16),
          jax.ShapeDtypeStruct((2, NROWS, D2), jnp.float32),
      ),
      mesh=mesh,
      compiler_params=pltpu.CompilerParams(use_tc_tiling_on_sc=False),
      scratch_types=[
          pltpu.VMEM((CF, EC), jnp.int32),
          pltpu.VMEM((CF, EC), jnp.int32),
          pltpu.VMEM((EC, IN), jnp.bfloat16),
          pltpu.VMEM((EC, IN), jnp.bfloat16),
          pltpu.VMEM((EC, D2), jnp.float32),
          pltpu.VMEM((EC, D2), jnp.float32),
          pltpu.VMEM_SHARED((NROWS, IN), jnp.bfloat16),
          pltpu.VMEM_SHARED((NROWS, D2), jnp.float32),
          pltpu.SemaphoreType.DMA,
          pltpu.SemaphoreType.DMA,
      ],
  )
  def seg(vals_hbm, src_hbm, dst_hbm, zeros_hbm, zd_hbm, ones_hbm,
          out_hbm, deg_hbm,
          src_v, dst_v, rows_a, rows_b, ones_v, zbuf, acc, acc_deg,
          sem_a, sem_b):
    cid = lax.axis_index("c")
    sid = lax.axis_index("s")
    start = jnp.where(cid == 0, sid * CF, 16 * CF + sid * CS)
    nhalf = jnp.where(cid == 0, CF // 2, CS // 2)
    base = sid * ROWS_PER_TILE
    with jax.named_scope("p1_zero"):
      pltpu.sync_copy(zeros_hbm, rows_a)
      pltpu.sync_copy(zd_hbm, zbuf)
      pltpu.sync_copy(ones_hbm, ones_v)

      def zbody(j, carry):
        pltpu.sync_copy(rows_a, acc.at[pl.ds(base + j * EC, EC)])
        pltpu.sync_copy(zbuf, acc_deg.at[pl.ds(base + j * EC, EC)])
        return carry

      lax.fori_loop(0, RCHUNKS, zbody, 0)
    with jax.named_scope("p1_idx"):
      pltpu.sync_copy(src_hbm.at[pl.ds(start, CF)], src_v)
      pltpu.sync_copy(dst_hbm.at[pl.ds(start, CF)], dst_v)
      plsc.subcore_barrier()

    def body(jj, carry):
      j0 = 2 * jj
      j1 = 2 * jj + 1
      ga = pltpu.async_copy(vals_hbm.at[src_v.at[j0]], rows_a, sem_a)
      gb = pltpu.async_copy(vals_hbm.at[src_v.at[j1]], rows_b, sem_b)
      pltpu.sync_copy(ones_v, acc_deg.at[dst_v.at[j0]], add=True)
      ga.wait()
      pltpu.sync_copy(rows_a, acc.at[dst_v.at[j0]], add=True)
      pltpu.sync_copy(ones_v, acc_deg.at[dst_v.at[j1]], add=True)
      gb.wait()
      pltpu.sync_copy(rows_b, acc.at[dst_v.at[j1]], add=True)
      return carry

    with jax.named_scope("p1_edges"):
      lax.fori_loop(0, nhalf, body, 0)
      plsc.subcore_barrier()

    def rbody(j, carry):
      pltpu.sync_copy(acc.at[pl.ds(base + j * EC, EC)], rows_a)
      pltpu.sync_copy(rows_a, out_hbm.at[cid, pl.ds(base + j * EC, EC)])
      pltpu.sync_copy(acc_deg.at[pl.ds(base + j * EC, EC)], zbuf)
      pltpu.sync_copy(zbuf, deg_hbm.at[cid, pl.ds(base + j * EC, EC)])
      return carry

    with jax.named_scope("p1_read"):
      lax.fori_loop(0, RCHUNKS, rbody, 0)

  return seg


def _sc_kernel_pass2():
  mesh = plsc.VectorSubcoreMesh(core_axis_name="c", subcore_axis_name="s")

  @functools.partial(
      pl.kernel,
      out_type=jax.ShapeDtypeStruct((2, NROWS, D2), jnp.float32),
      mesh=mesh,
      compiler_params=pltpu.CompilerParams(use_tc_tiling_on_sc=False),
      scratch_types=[
          pltpu.VMEM((CF, EC), jnp.int32),
          pltpu.VMEM((CF, EC), jnp.int32),
          pltpu.VMEM((EC, D2), jnp.float32),
          pltpu.VMEM((EC, D2), jnp.float32),
          pltpu.VMEM_SHARED((NROWS, D2), jnp.float32),
          pltpu.SemaphoreType.DMA,
          pltpu.SemaphoreType.DMA,
      ],
  )
  def seg(vals_hbm, src_hbm, dst_hbm, zd_hbm, out_hbm,
          src_v, dst_v, rows_a, rows_b, acc, sem_a, sem_b):
    cid = lax.axis_index("c")
    sid = lax.axis_index("s")
    start = jnp.where(cid == 0, sid * CF, 16 * CF + sid * CS)
    nhalf = jnp.where(cid == 0, CF // 2, CS // 2)
    base = sid * ROWS_PER_TILE
    pltpu.sync_copy(zd_hbm, rows_a)

    def zbody(j, carry):
      pltpu.sync_copy(rows_a, acc.at[pl.ds(base + j * EC, EC)])
      return carry

    lax.fori_loop(0, RCHUNKS, zbody, 0)
    pltpu.sync_copy(src_hbm.at[pl.ds(start, CF)], src_v)
    pltpu.sync_copy(dst_hbm.at[pl.ds(start, CF)], dst_v)
    plsc.subcore_barrier()

    def body(jj, carry):
      j0 = 2 * jj
      j1 = 2 * jj + 1
      ga = pltpu.async_copy(vals_hbm.at[src_v.at[j0]], rows_a, sem_a)
      gb = pltpu.async_copy(vals_hbm.at[src_v.at[j1]], rows_b, sem_b)
      ga.wait()
      pltpu.sync_copy(rows_a, acc.at[dst_v.at[j0]], add=True)
      gb.wait()
      pltpu.sync_copy(rows_b, acc.at[dst_v.at[j1]], add=True)
      return carry

    lax.fori_loop(0, nhalf, body, 0)
    plsc.subcore_barrier()

    def rbody(j, carry):
      pltpu.sync_copy(acc.at[pl.ds(base + j * EC, EC)], rows_a)
      pltpu.sync_copy(rows_a, out_hbm.at[cid, pl.ds(base + j * EC, EC)])
      return carry

    lax.fori_loop(0, RCHUNKS, rbody, 0)

  return seg


_sc_cache = {}


def _sc_pass1(*args):
  if 1 not in _sc_cache:
    _sc_cache[1] = _sc_kernel_pass1()
  return _sc_cache[1](*args)


def _sc_pass2(*args):
  if 2 not in _sc_cache:
    _sc_cache[2] = _sc_kernel_pass2()
  return _sc_cache[2](*args)


_BLK = 1024


def _dense1_body(part, degp, x, w1, w2, w3, b1, b3, wc2, b12, b32,
                 p_out, r_out):
  gx = part[0].astype(jnp.float32) + part[1].astype(jnp.float32)
  deg = degp[0, :, 0:1] + degp[1, :, 0:1]
  h = jnp.dot(gx, w1[:], preferred_element_type=jnp.float32)
  h = h + jnp.dot(x[:] * (-deg), w2[:], preferred_element_type=jnp.float32)
  h = h + jnp.dot(x[:], w3[:], preferred_element_type=jnp.float32)
  h = h + deg * b1[:] + b3[:]
  h = jnp.maximum(h, 0.0)
  m2 = jnp.dot(h, wc2[:], preferred_element_type=jnp.float32)
  p = m2[:, 0:OUT]
  p_out[:] = jnp.concatenate([p, jnp.zeros_like(p)], axis=1)
  r_out[:] = deg * b12[:] - deg * m2[:, OUT:2 * OUT] + m2[:, 2 * OUT:3 * OUT] \
      + b32[:]


def _dense2_body(gp, r, o):
  s = gp[0, :, 0:OUT] + gp[1, :, 0:OUT] + r[:]
  o[:] = jnp.maximum(s, 0.0)


def kernel(x, edge_index, W1_1, b1_1, W2_1, W3_1, b3_1,
           W1_2, b1_2, W2_2, W3_2, b3_2):
  src = edge_index[0]
  dst = edge_index[1]
  pad = EPAD - E
  src_p = jnp.concatenate([src, jnp.zeros((pad,), jnp.int32)])
  dst_p = jnp.concatenate([dst, jnp.full((pad,), NROWS - 1, jnp.int32)])
  src_p = src_p.reshape(EPAD // EC, EC)
  dst_p = dst_p.reshape(EPAD // EC, EC)

  zeros1 = jnp.zeros((EC, IN), jnp.bfloat16)
  zerosd = jnp.zeros((EC, D2), jnp.float32)
  ones1 = jnp.ones((EC, D2), jnp.float32)
  part1, degp = _sc_pass1(x.astype(jnp.bfloat16), src_p, dst_p,
                          zeros1, zerosd, ones1)

  wc2 = jnp.concatenate([W1_2, W2_2, W3_2], axis=1)
  grid = NROWS // _BLK
  p_pad, r = pl.pallas_call(
      _dense1_body,
      grid=(grid,),
      in_specs=[
          pl.BlockSpec((2, _BLK, IN), lambda i: (0, i, 0)),
          pl.BlockSpec((2, _BLK, D2), lambda i: (0, i, 0)),
          pl.BlockSpec((_BLK, IN), lambda i: (i, 0)),
          pl.BlockSpec((IN, HID), lambda i: (0, 0)),
          pl.BlockSpec((IN, HID), lambda i: (0, 0)),
          pl.BlockSpec((IN, HID), lambda i: (0, 0)),
          pl.BlockSpec((1, HID), lambda i: (0, 0)),
          pl.BlockSpec((1, HID), lambda i: (0, 0)),
          pl.BlockSpec((HID, 3 * OUT), lambda i: (0, 0)),
          pl.BlockSpec((1, OUT), lambda i: (0, 0)),
          pl.BlockSpec((1, OUT), lambda i: (0, 0)),
      ],
      out_specs=[
          pl.BlockSpec((_BLK, D2), lambda i: (i, 0)),
          pl.BlockSpec((_BLK, OUT), lambda i: (i, 0)),
      ],
      out_shape=[
          jax.ShapeDtypeStruct((NROWS, D2), jnp.float32),
          jax.ShapeDtypeStruct((NROWS, OUT), jnp.float32),
      ],
  )(part1, degp, _pad_rows(x), W1_1, W2_1, W3_1, b1_1.reshape(1, HID),
    b3_1.reshape(1, HID), wc2, b1_2.reshape(1, OUT), b3_2.reshape(1, OUT))

  part2 = _sc_pass2(p_pad, src_p, dst_p, zerosd)

  out = pl.pallas_call(
      _dense2_body,
      grid=(grid,),
      in_specs=[
          pl.BlockSpec((2, _BLK, D2), lambda i: (0, i, 0)),
          pl.BlockSpec((_BLK, OUT), lambda i: (i, 0)),
      ],
      out_specs=pl.BlockSpec((_BLK, OUT), lambda i: (i, 0)),
      out_shape=jax.ShapeDtypeStruct((NROWS, OUT), jnp.float32),
  )(part2, r)
  return out[:N]


def _pad_rows(x):
  return jnp.concatenate(
      [x, jnp.zeros((NROWS - N, x.shape[1]), x.dtype)], axis=0)


# speedup vs baseline: 1.8549x; 1.3199x over previous
import functools

import jax
import jax.numpy as jnp
from jax import lax
from jax.experimental import pallas as pl
from jax.experimental.pallas import tpu as pltpu
from jax.experimental.pallas import tpu_sc as plsc

N = 10000
E = 160000
IN = 128
OUT = 8
HID = 800

EC = 128
NCHUNKS = E // EC
NROWS = 10240
ROWS_PER_TILE = NROWS // 16
RC = 64
RCHUNKS = ROWS_PER_TILE // RC
D2 = 16
CF1, CS1 = 58, 20
CF2, CS2 = 48, 30
IDXROWS = 64


def _tile_plan(cid, sid, cf, cs):
  fast_start = sid * cf
  slow_start = 16 * cf + sid * cs + jnp.minimum(sid, 2)
  start = jnp.where(cid == 0, fast_start, slow_start)
  stage = jnp.minimum(start, NCHUNKS - IDXROWS)
  off = start - stage
  npairs = jnp.where(cid == 0, cf // 2, cs // 2)
  extra = jnp.logical_and(cid == 1, sid < 2)
  return stage, off, npairs, extra


def _sc_kernel_pass1():
  mesh = plsc.VectorSubcoreMesh(core_axis_name="c", subcore_axis_name="s")

  @functools.partial(
      pl.kernel,
      out_type=(
          jax.ShapeDtypeStruct((2, NROWS, IN), jnp.bfloat16),
          jax.ShapeDtypeStruct((2, NROWS, D2), jnp.float32),
      ),
      mesh=mesh,
      compiler_params=pltpu.CompilerParams(use_tc_tiling_on_sc=False),
      scratch_types=[
          pltpu.VMEM((IDXROWS, EC), jnp.int32),
          pltpu.VMEM((IDXROWS, EC), jnp.int32),
          pltpu.VMEM((EC, IN), jnp.bfloat16),
          pltpu.VMEM((EC, IN), jnp.bfloat16),
          pltpu.VMEM((EC, D2), jnp.float32),
          pltpu.VMEM((RC, D2), jnp.float32),
          pltpu.VMEM_SHARED((NROWS, IN), jnp.bfloat16),
          pltpu.VMEM_SHARED((NROWS, D2), jnp.float32),
          pltpu.SemaphoreType.DMA,
          pltpu.SemaphoreType.DMA,
      ],
  )
  def seg(vals_hbm, src_hbm, dst_hbm, zeros_hbm, zd_hbm, ones_hbm,
          out_hbm, deg_hbm,
          src_v, dst_v, rows_a, rows_b, ones_v, zbuf, acc, acc_deg,
          sem_a, sem_b):
    cid = lax.axis_index("c")
    sid = lax.axis_index("s")
    stage, off, npairs, extra = _tile_plan(cid, sid, CF1, CS1)
    base = sid * ROWS_PER_TILE
    with jax.named_scope("p1_zero"):
      pltpu.sync_copy(zeros_hbm, rows_a)
      pltpu.sync_copy(zd_hbm, zbuf)
      pltpu.sync_copy(ones_hbm, ones_v)

      def zbody(j, carry):
        pltpu.sync_copy(rows_a.at[pl.ds(0, RC)],
                        acc.at[pl.ds(base + j * RC, RC)])
        pltpu.sync_copy(zbuf, acc_deg.at[pl.ds(base + j * RC, RC)])
        return carry

      lax.fori_loop(0, RCHUNKS, zbody, 0)
    with jax.named_scope("p1_idx"):
      pltpu.sync_copy(src_hbm.at[pl.ds(stage, IDXROWS)], src_v)
      pltpu.sync_copy(dst_hbm.at[pl.ds(stage, IDXROWS)], dst_v)
      plsc.subcore_barrier()

    def body(jj, carry):
      j0 = off + 2 * jj
      j1 = off + 2 * jj + 1
      ga = pltpu.async_copy(vals_hbm.at[src_v.at[j0]], rows_a, sem_a)
      gb = pltpu.async_copy(vals_hbm.at[src_v.at[j1]], rows_b, sem_b)
      pltpu.sync_copy(ones_v, acc_deg.at[dst_v.at[j0]], add=True)
      ga.wait()
      pltpu.sync_copy(rows_a, acc.at[dst_v.at[j0]], add=True)
      pltpu.sync_copy(ones_v, acc_deg.at[dst_v.at[j1]], add=True)
      gb.wait()
      pltpu.sync_copy(rows_b, acc.at[dst_v.at[j1]], add=True)
      return carry

    with jax.named_scope("p1_edges"):
      lax.fori_loop(0, npairs, body, 0)

      @pl.when(extra)
      def _():
        j = off + 2 * npairs
        pltpu.async_copy(vals_hbm.at[src_v.at[j]], rows_a, sem_a).wait()
        pltpu.sync_copy(ones_v, acc_deg.at[dst_v.at[j]], add=True)
        pltpu.sync_copy(rows_a, acc.at[dst_v.at[j]], add=True)

      plsc.subcore_barrier()

    def rbody(j, carry):
      pltpu.sync_copy(acc.at[pl.ds(base + j * RC, RC)],
                      rows_a.at[pl.ds(0, RC)])
      pltpu.sync_copy(rows_a.at[pl.ds(0, RC)],
                      out_hbm.at[cid, pl.ds(base + j * RC, RC)])
      pltpu.sync_copy(acc_deg.at[pl.ds(base + j * RC, RC)], zbuf)
      pltpu.sync_copy(zbuf, deg_hbm.at[cid, pl.ds(base + j * RC, RC)])
      return carry

    with jax.named_scope("p1_read"):
      lax.fori_loop(0, RCHUNKS, rbody, 0)

  return seg


def _sc_kernel_pass2():
  mesh = plsc.VectorSubcoreMesh(core_axis_name="c", subcore_axis_name="s")

  @functools.partial(
      pl.kernel,
      out_type=jax.ShapeDtypeStruct((2, NROWS, D2), jnp.float32),
      mesh=mesh,
      compiler_params=pltpu.CompilerParams(use_tc_tiling_on_sc=False),
      scratch_types=[
          pltpu.VMEM((IDXROWS, EC), jnp.int32),
          pltpu.VMEM((IDXROWS, EC), jnp.int32),
          pltpu.VMEM((EC, D2), jnp.float32),
          pltpu.VMEM((EC, D2), jnp.float32),
          pltpu.VMEM_SHARED((NROWS, D2), jnp.float32),
          pltpu.SemaphoreType.DMA,
          pltpu.SemaphoreType.DMA,
      ],
  )
  def seg(vals_hbm, src_hbm, dst_hbm, zd_hbm, out_hbm,
          src_v, dst_v, rows_a, rows_b, acc, sem_a, sem_b):
    cid = lax.axis_index("c")
    sid = lax.axis_index("s")
    stage, off, npairs, extra = _tile_plan(cid, sid, CF2, CS2)
    base = sid * ROWS_PER_TILE
    pltpu.sync_copy(zd_hbm, rows_a.at[pl.ds(0, RC)])

    def zbody(j, carry):
      pltpu.sync_copy(rows_a.at[pl.ds(0, RC)],
                      acc.at[pl.ds(base + j * RC, RC)])
      return carry

    lax.fori_loop(0, RCHUNKS, zbody, 0)
    pltpu.sync_copy(src_hbm.at[pl.ds(stage, IDXROWS)], src_v)
    pltpu.sync_copy(dst_hbm.at[pl.ds(stage, IDXROWS)], dst_v)
    plsc.subcore_barrier()

    def body(jj, carry):
      j0 = off + 2 * jj
      j1 = off + 2 * jj + 1
      ga = pltpu.async_copy(vals_hbm.at[src_v.at[j0]], rows_a, sem_a)
      gb = pltpu.async_copy(vals_hbm.at[src_v.at[j1]], rows_b, sem_b)
      ga.wait()
      pltpu.sync_copy(rows_a, acc.at[dst_v.at[j0]], add=True)
      gb.wait()
      pltpu.sync_copy(rows_b, acc.at[dst_v.at[j1]], add=True)
      return carry

    lax.fori_loop(0, npairs, body, 0)

    @pl.when(extra)
    def _():
      j = off + 2 * npairs
      pltpu.async_copy(vals_hbm.at[src_v.at[j]], rows_a, sem_a).wait()
      pltpu.sync_copy(rows_a, acc.at[dst_v.at[j]], add=True)

    plsc.subcore_barrier()

    def rbody(j, carry):
      pltpu.sync_copy(acc.at[pl.ds(base + j * RC, RC)],
                      rows_a.at[pl.ds(0, RC)])
      pltpu.sync_copy(rows_a.at[pl.ds(0, RC)],
                      out_hbm.at[cid, pl.ds(base + j * RC, RC)])
      return carry

    lax.fori_loop(0, RCHUNKS, rbody, 0)

  return seg


_sc_cache = {}


def _sc_pass1(*args):
  if 1 not in _sc_cache:
    _sc_cache[1] = _sc_kernel_pass1()
  return _sc_cache[1](*args)


def _sc_pass2(*args):
  if 2 not in _sc_cache:
    _sc_cache[2] = _sc_kernel_pass2()
  return _sc_cache[2](*args)


_BLK = 1024


def _dense1_body(part, degp, x, w1, w2, w3, b1, b3, wc2, b12, b32,
                 p_out, r_out):
  gx = part[0].astype(jnp.float32) + part[1].astype(jnp.float32)
  deg = degp[0, :, 0:1] + degp[1, :, 0:1]
  h = jnp.dot(gx, w1[:], preferred_element_type=jnp.float32)
  h = h + jnp.dot(x[:] * (-deg), w2[:], preferred_element_type=jnp.float32)
  h = h + jnp.dot(x[:], w3[:], preferred_element_type=jnp.float32)
  h = h + deg * b1[:] + b3[:]
  h = jnp.maximum(h, 0.0)
  m2 = jnp.dot(h, wc2[:], preferred_element_type=jnp.float32)
  p = m2[:, 0:OUT]
  p_out[:] = jnp.concatenate([p, jnp.zeros_like(p)], axis=1)
  r_out[:] = deg * b12[:] - deg * m2[:, OUT:2 * OUT] + m2[:, 2 * OUT:3 * OUT] \
      + b32[:]


def _dense2_body(gp, r, o):
  s = gp[0, :, 0:OUT] + gp[1, :, 0:OUT] + r[:]
  o[:] = jnp.maximum(s, 0.0)


def kernel(x, edge_index, W1_1, b1_1, W2_1, W3_1, b3_1,
           W1_2, b1_2, W2_2, W3_2, b3_2):
  src_p = edge_index[0].reshape(NCHUNKS, EC)
  dst_p = edge_index[1].reshape(NCHUNKS, EC)

  zeros1 = jnp.zeros((EC, IN), jnp.bfloat16)
  zerosd = jnp.zeros((RC, D2), jnp.float32)
  ones1 = jnp.ones((EC, D2), jnp.float32)
  part1, degp = _sc_pass1(x.astype(jnp.bfloat16), src_p, dst_p,
                          zeros1, zerosd, ones1)

  wc2 = jnp.concatenate([W1_2, W2_2, W3_2], axis=1)
  grid = NROWS // _BLK
  p_pad, r = pl.pallas_call(
      _dense1_body,
      grid=(grid,),
      in_specs=[
          pl.BlockSpec((2, _BLK, IN), lambda i: (0, i, 0)),
          pl.BlockSpec((2, _BLK, D2), lambda i: (0, i, 0)),
          pl.BlockSpec((_BLK, IN), lambda i: (i, 0)),
          pl.BlockSpec((IN, HID), lambda i: (0, 0)),
          pl.BlockSpec((IN, HID), lambda i: (0, 0)),
          pl.BlockSpec((IN, HID), lambda i: (0, 0)),
          pl.BlockSpec((1, HID), lambda i: (0, 0)),
          pl.BlockSpec((1, HID), lambda i: (0, 0)),
          pl.BlockSpec((HID, 3 * OUT), lambda i: (0, 0)),
          pl.BlockSpec((1, OUT), lambda i: (0, 0)),
          pl.BlockSpec((1, OUT), lambda i: (0, 0)),
      ],
      out_specs=[
          pl.BlockSpec((_BLK, D2), lambda i: (i, 0)),
          pl.BlockSpec((_BLK, OUT), lambda i: (i, 0)),
      ],
      out_shape=[
          jax.ShapeDtypeStruct((NROWS, D2), jnp.float32),
          jax.ShapeDtypeStruct((NROWS, OUT), jnp.float32),
      ],
  )(part1, degp, _pad_rows(x), W1_1, W2_1, W3_1, b1_1.reshape(1, HID),
    b3_1.reshape(1, HID), wc2, b1_2.reshape(1, OUT), b3_2.reshape(1, OUT))

  part2 = _sc_pass2(p_pad, src_p, dst_p, zerosd)

  out = pl.pallas_call(
      _dense2_body,
      grid=(grid,),
      in_specs=[
          pl.BlockSpec((2, _BLK, D2), lambda i: (0, i, 0)),
          pl.BlockSpec((_BLK, OUT), lambda i: (i, 0)),
      ],
      out_specs=pl.BlockSpec((_BLK, OUT), lambda i: (i, 0)),
      out_shape=jax.ShapeDtypeStruct((NROWS, OUT), jnp.float32),
  )(part2, r)
  return out[:N]


def _pad_rows(x):
  return jnp.concatenate(
      [x, jnp.zeros((NROWS - N, x.shape[1]), x.dtype)], axis=0)


# speedup vs baseline: 2.0523x; 1.1064x over previous
import functools

import jax
import jax.numpy as jnp
from jax import lax
from jax.experimental import pallas as pl
from jax.experimental.pallas import tpu as pltpu
from jax.experimental.pallas import tpu_sc as plsc

N = 10000
E = 160000
IN = 128
OUT = 8
HID = 800

EC = 128
NCHUNKS = E // EC
NROWS = 10240
ROWS_PER_TILE = NROWS // 16
RC = 64
RCHUNKS = ROWS_PER_TILE // RC
D2 = 16
CPT = 39
IDXROWS = 64


def _tile_plan(cid, sid):
  bump = jnp.where(cid == 0, 0, jnp.minimum(sid, 2))
  start = (cid * 16 + sid) * CPT + bump
  stage = jnp.minimum(start, NCHUNKS - IDXROWS)
  off = start - stage
  has40 = jnp.logical_and(cid == 1, sid < 2)
  npairs = jnp.where(has40, CPT // 2 + 1, CPT // 2)
  extra = jnp.logical_not(has40)
  return stage, off, npairs, extra


def _sc_kernel_pass1():
  mesh = plsc.VectorSubcoreMesh(core_axis_name="c", subcore_axis_name="s")

  @functools.partial(
      pl.kernel,
      out_type=(
          jax.ShapeDtypeStruct((2, NROWS, IN), jnp.bfloat16),
          jax.ShapeDtypeStruct((2, NROWS, D2), jnp.float32),
      ),
      mesh=mesh,
      compiler_params=pltpu.CompilerParams(use_tc_tiling_on_sc=False),
      scratch_types=[
          pltpu.VMEM((IDXROWS, EC), jnp.int32),
          pltpu.VMEM((IDXROWS, EC), jnp.int32),
          pltpu.VMEM((EC, IN), jnp.bfloat16),
          pltpu.VMEM((EC, IN), jnp.bfloat16),
          pltpu.VMEM((EC, D2), jnp.float32),
          pltpu.VMEM((RC, D2), jnp.float32),
          pltpu.VMEM_SHARED((NROWS, IN), jnp.bfloat16),
          pltpu.VMEM_SHARED((NROWS, D2), jnp.float32),
          pltpu.SemaphoreType.DMA,
          pltpu.SemaphoreType.DMA,
      ],
  )
  def seg(vals_hbm, src_hbm, dst_hbm, zeros_hbm, zd_hbm, ones_hbm,
          out_hbm, deg_hbm,
          src_v, dst_v, rows_a, rows_b, ones_v, zbuf, acc, acc_deg,
          sem_a, sem_b):
    cid = lax.axis_index("c")
    sid = lax.axis_index("s")
    stage, off, npairs, extra = _tile_plan(cid, sid)
    base = sid * ROWS_PER_TILE
    with jax.named_scope("p1_zero"):
      pltpu.sync_copy(zeros_hbm, rows_a)
      pltpu.sync_copy(zd_hbm, zbuf)
      pltpu.sync_copy(ones_hbm, ones_v)

      def zbody(j, carry):
        pltpu.sync_copy(rows_a.at[pl.ds(0, RC)],
                        acc.at[pl.ds(base + j * RC, RC)])
        pltpu.sync_copy(zbuf, acc_deg.at[pl.ds(base + j * RC, RC)])
        return carry

      lax.fori_loop(0, RCHUNKS, zbody, 0)
    with jax.named_scope("p1_idx"):
      pltpu.sync_copy(src_hbm.at[pl.ds(stage, IDXROWS)], src_v)
      pltpu.sync_copy(dst_hbm.at[pl.ds(stage, IDXROWS)], dst_v)
      plsc.subcore_barrier()

    def body(jj, carry):
      j0 = off + 2 * jj
      j1 = off + 2 * jj + 1
      ga = pltpu.async_copy(vals_hbm.at[src_v.at[j0]], rows_a, sem_a)
      gb = pltpu.async_copy(vals_hbm.at[src_v.at[j1]], rows_b, sem_b)
      pltpu.sync_copy(ones_v, acc_deg.at[dst_v.at[j0]], add=True)
      ga.wait()
      pltpu.sync_copy(rows_a, acc.at[dst_v.at[j0]], add=True)
      pltpu.sync_copy(ones_v, acc_deg.at[dst_v.at[j1]], add=True)
      gb.wait()
      pltpu.sync_copy(rows_b, acc.at[dst_v.at[j1]], add=True)
      return carry

    with jax.named_scope("p1_edges"):
      lax.fori_loop(0, npairs, body, 0)

      @pl.when(extra)
      def _():
        j = off + 2 * npairs
        pltpu.async_copy(vals_hbm.at[src_v.at[j]], rows_a, sem_a).wait()
        pltpu.sync_copy(ones_v, acc_deg.at[dst_v.at[j]], add=True)
        pltpu.sync_copy(rows_a, acc.at[dst_v.at[j]], add=True)

      plsc.subcore_barrier()

    def rbody(j, carry):
      pltpu.sync_copy(acc.at[pl.ds(base + j * RC, RC)],
                      rows_a.at[pl.ds(0, RC)])
      pltpu.sync_copy(rows_a.at[pl.ds(0, RC)],
                      out_hbm.at[cid, pl.ds(base + j * RC, RC)])
      pltpu.sync_copy(acc_deg.at[pl.ds(base + j * RC, RC)], zbuf)
      pltpu.sync_copy(zbuf, deg_hbm.at[cid, pl.ds(base + j * RC, RC)])
      return carry

    with jax.named_scope("p1_read"):
      lax.fori_loop(0, RCHUNKS, rbody, 0)

  return seg


def _sc_kernel_pass2():
  mesh = plsc.VectorSubcoreMesh(core_axis_name="c", subcore_axis_name="s")

  @functools.partial(
      pl.kernel,
      out_type=jax.ShapeDtypeStruct((2, NROWS, D2), jnp.float32),
      mesh=mesh,
      compiler_params=pltpu.CompilerParams(use_tc_tiling_on_sc=False),
      scratch_types=[
          pltpu.VMEM((IDXROWS, EC), jnp.int32),
          pltpu.VMEM((IDXROWS, EC), jnp.int32),
          pltpu.VMEM((EC, D2), jnp.float32),
          pltpu.VMEM((EC, D2), jnp.float32),
          pltpu.VMEM_SHARED((NROWS, D2), jnp.float32),
          pltpu.SemaphoreType.DMA,
          pltpu.SemaphoreType.DMA,
      ],
  )
  def seg(vals_hbm, src_hbm, dst_hbm, zd_hbm, out_hbm,
          src_v, dst_v, rows_a, rows_b, acc, sem_a, sem_b):
    cid = lax.axis_index("c")
    sid = lax.axis_index("s")
    stage, off, npairs, extra = _tile_plan(cid, sid)
    base = sid * ROWS_PER_TILE
    pltpu.sync_copy(zd_hbm, rows_a.at[pl.ds(0, RC)])

    def zbody(j, carry):
      pltpu.sync_copy(rows_a.at[pl.ds(0, RC)],
                      acc.at[pl.ds(base + j * RC, RC)])
      return carry

    lax.fori_loop(0, RCHUNKS, zbody, 0)
    pltpu.sync_copy(src_hbm.at[pl.ds(stage, IDXROWS)], src_v)
    pltpu.sync_copy(dst_hbm.at[pl.ds(stage, IDXROWS)], dst_v)
    plsc.subcore_barrier()

    def body(jj, carry):
      j0 = off + 2 * jj
      j1 = off + 2 * jj + 1
      ga = pltpu.async_copy(vals_hbm.at[src_v.at[j0]], rows_a, sem_a)
      gb = pltpu.async_copy(vals_hbm.at[src_v.at[j1]], rows_b, sem_b)
      ga.wait()
      pltpu.sync_copy(rows_a, acc.at[dst_v.at[j0]], add=True)
      gb.wait()
      pltpu.sync_copy(rows_b, acc.at[dst_v.at[j1]], add=True)
      return carry

    lax.fori_loop(0, npairs, body, 0)

    @pl.when(extra)
    def _():
      j = off + 2 * npairs
      pltpu.async_copy(vals_hbm.at[src_v.at[j]], rows_a, sem_a).wait()
      pltpu.sync_copy(rows_a, acc.at[dst_v.at[j]], add=True)

    plsc.subcore_barrier()

    def rbody(j, carry):
      pltpu.sync_copy(acc.at[pl.ds(base + j * RC, RC)],
                      rows_a.at[pl.ds(0, RC)])
      pltpu.sync_copy(rows_a.at[pl.ds(0, RC)],
                      out_hbm.at[cid, pl.ds(base + j * RC, RC)])
      return carry

    lax.fori_loop(0, RCHUNKS, rbody, 0)

  return seg


_sc_cache = {}


def _sc_pass1(*args):
  if 1 not in _sc_cache:
    _sc_cache[1] = _sc_kernel_pass1()
  return _sc_cache[1](*args)


def _sc_pass2(*args):
  if 2 not in _sc_cache:
    _sc_cache[2] = _sc_kernel_pass2()
  return _sc_cache[2](*args)


_BLK = 1000


def _dense1_body(part, degp, x, w1, w2, w3, b1, b3, wc2, b12, b32,
                 p_out, r_out):
  gx = part[0] + part[1]
  deg = degp[0, :, 0:1] + degp[1, :, 0:1]
  xd = (x[:].astype(jnp.float32) * (-deg)).astype(jnp.bfloat16)
  h = jnp.dot(gx, w1[:], preferred_element_type=jnp.float32)
  h = h + jnp.dot(xd, w2[:], preferred_element_type=jnp.float32)
  h = h + jnp.dot(x[:], w3[:], preferred_element_type=jnp.float32)
  h = h + deg * b1[:] + b3[:]
  h = jnp.maximum(h, 0.0)
  m2 = jnp.dot(h.astype(jnp.bfloat16), wc2[:],
               preferred_element_type=jnp.float32)
  p = m2[:, 0:OUT]
  p_out[:] = jnp.concatenate([p, jnp.zeros_like(p)], axis=1)
  r_out[:] = deg * b12[:] - deg * m2[:, OUT:2 * OUT] + m2[:, 2 * OUT:3 * OUT] \
      + b32[:]


def _dense2_body(gp, r, o):
  s = gp[0, :, 0:OUT] + gp[1, :, 0:OUT] + r[:]
  o[:] = jnp.maximum(s, 0.0)


def kernel(x, edge_index, W1_1, b1_1, W2_1, W3_1, b3_1,
           W1_2, b1_2, W2_2, W3_2, b3_2):
  src_p = edge_index[0].reshape(NCHUNKS, EC)
  dst_p = edge_index[1].reshape(NCHUNKS, EC)

  xb = x.astype(jnp.bfloat16)
  zeros1 = jnp.zeros((EC, IN), jnp.bfloat16)
  zerosd = jnp.zeros((RC, D2), jnp.float32)
  ones1 = jnp.ones((EC, D2), jnp.float32)
  part1, degp = _sc_pass1(xb, src_p, dst_p, zeros1, zerosd, ones1)

  wc2 = jnp.concatenate([W1_2, W2_2, W3_2],
                        axis=1).astype(jnp.bfloat16)
  grid = N // _BLK
  p_pad, r = pl.pallas_call(
      _dense1_body,
      grid=(grid,),
      in_specs=[
          pl.BlockSpec((2, _BLK, IN), lambda i: (0, i, 0)),
          pl.BlockSpec((2, _BLK, D2), lambda i: (0, i, 0)),
          pl.BlockSpec((_BLK, IN), lambda i: (i, 0)),
          pl.BlockSpec((IN, HID), lambda i: (0, 0)),
          pl.BlockSpec((IN, HID), lambda i: (0, 0)),
          pl.BlockSpec((IN, HID), lambda i: (0, 0)),
          pl.BlockSpec((1, HID), lambda i: (0, 0)),
          pl.BlockSpec((1, HID), lambda i: (0, 0)),
          pl.BlockSpec((HID, 3 * OUT), lambda i: (0, 0)),
          pl.BlockSpec((1, OUT), lambda i: (0, 0)),
          pl.BlockSpec((1, OUT), lambda i: (0, 0)),
      ],
      out_specs=[
          pl.BlockSpec((_BLK, D2), lambda i: (i, 0)),
          pl.BlockSpec((_BLK, OUT), lambda i: (i, 0)),
      ],
      out_shape=[
          jax.ShapeDtypeStruct((N, D2), jnp.float32),
          jax.ShapeDtypeStruct((N, OUT), jnp.float32),
      ],
  )(part1, degp, xb, W1_1.astype(jnp.bfloat16), W2_1.astype(jnp.bfloat16),
    W3_1.astype(jnp.bfloat16), b1_1.reshape(1, HID), b3_1.reshape(1, HID),
    wc2, b1_2.reshape(1, OUT), b3_2.reshape(1, OUT))

  part2 = _sc_pass2(p_pad, src_p, dst_p, zerosd)

  out = pl.pallas_call(
      _dense2_body,
      grid=(grid,),
      in_specs=[
          pl.BlockSpec((2, _BLK, D2), lambda i: (0, i, 0)),
          pl.BlockSpec((_BLK, OUT), lambda i: (i, 0)),
      ],
      out_specs=pl.BlockSpec((_BLK, OUT), lambda i: (i, 0)),
      out_shape=jax.ShapeDtypeStruct((N, OUT), jnp.float32),
  )(part2, r)
  return out


# speedup vs baseline: 2.0923x; 1.0195x over previous
import functools

import jax
import jax.numpy as jnp
from jax import lax
from jax.experimental import pallas as pl
from jax.experimental.pallas import tpu as pltpu
from jax.experimental.pallas import tpu_sc as plsc

N = 10000
E = 160000
IN = 128
OUT = 8
HID = 800

EC = 128
NCHUNKS = E // EC
NROWS = 10240
ROWS_PER_TILE = NROWS // 16
RC = 64
RCHUNKS = ROWS_PER_TILE // RC
D2 = 16
CPT = 39
IDXROWS = 64


def _tile_plan(cid, sid):
  bump = jnp.where(cid == 0, 0, jnp.minimum(sid, 2))
  start = (cid * 16 + sid) * CPT + bump
  stage = jnp.minimum(start, NCHUNKS - IDXROWS)
  off = start - stage
  has40 = jnp.logical_and(cid == 1, sid < 2)
  npairs = jnp.where(has40, CPT // 2 + 1, CPT // 2)
  extra = jnp.logical_not(has40)
  return stage, off, npairs, extra


def _sc_kernel_pass1():
  mesh = plsc.VectorSubcoreMesh(core_axis_name="c", subcore_axis_name="s")

  @functools.partial(
      pl.kernel,
      out_type=(
          jax.ShapeDtypeStruct((2, NROWS, IN), jnp.bfloat16),
          jax.ShapeDtypeStruct((2, NROWS, D2), jnp.float32),
      ),
      mesh=mesh,
      compiler_params=pltpu.CompilerParams(use_tc_tiling_on_sc=False),
      scratch_types=[
          pltpu.VMEM((IDXROWS, EC), jnp.int32),
          pltpu.VMEM((IDXROWS, EC), jnp.int32),
          pltpu.VMEM((EC, IN), jnp.bfloat16),
          pltpu.VMEM((EC, IN), jnp.bfloat16),
          pltpu.VMEM((EC, D2), jnp.float32),
          pltpu.VMEM((RC, D2), jnp.float32),
          pltpu.VMEM_SHARED((NROWS, IN), jnp.bfloat16),
          pltpu.VMEM_SHARED((NROWS, D2), jnp.float32),
          pltpu.SemaphoreType.DMA,
          pltpu.SemaphoreType.DMA,
      ],
  )
  def seg(vals_hbm, src_hbm, dst_hbm, zeros_hbm, zd_hbm, ones_hbm,
          out_hbm, deg_hbm,
          src_v, dst_v, rows_a, rows_b, ones_v, zbuf, acc, acc_deg,
          sem_a, sem_b):
    cid = lax.axis_index("c")
    sid = lax.axis_index("s")
    stage, off, npairs, extra = _tile_plan(cid, sid)
    base = sid * ROWS_PER_TILE
    with jax.named_scope("p1_zero"):
      pltpu.sync_copy(zeros_hbm, rows_a)
      pltpu.sync_copy(zd_hbm, zbuf)
      pltpu.sync_copy(ones_hbm, ones_v)

      def zbody(j, carry):
        pltpu.sync_copy(rows_a.at[pl.ds(0, RC)],
                        acc.at[pl.ds(base + j * RC, RC)])
        pltpu.sync_copy(zbuf, acc_deg.at[pl.ds(base + j * RC, RC)])
        return carry

      lax.fori_loop(0, RCHUNKS, zbody, 0)
    with jax.named_scope("p1_idx"):
      pltpu.sync_copy(src_hbm.at[pl.ds(stage, IDXROWS)], src_v)
      pltpu.sync_copy(dst_hbm.at[pl.ds(stage, IDXROWS)], dst_v)
      plsc.subcore_barrier()

    def body(jj, carry):
      j0 = off + 2 * jj
      j1 = off + 2 * jj + 1
      ga = pltpu.async_copy(vals_hbm.at[src_v.at[j0]], rows_a, sem_a)
      gb = pltpu.async_copy(vals_hbm.at[src_v.at[j1]], rows_b, sem_b)
      pltpu.sync_copy(ones_v, acc_deg.at[dst_v.at[j0]], add=True)
      ga.wait()
      pltpu.sync_copy(rows_a, acc.at[dst_v.at[j0]], add=True)
      pltpu.sync_copy(ones_v, acc_deg.at[dst_v.at[j1]], add=True)
      gb.wait()
      pltpu.sync_copy(rows_b, acc.at[dst_v.at[j1]], add=True)
      return carry

    with jax.named_scope("p1_edges"):
      lax.fori_loop(0, npairs, body, 0)

      @pl.when(extra)
      def _():
        j = off + 2 * npairs
        pltpu.async_copy(vals_hbm.at[src_v.at[j]], rows_a, sem_a).wait()
        pltpu.sync_copy(ones_v, acc_deg.at[dst_v.at[j]], add=True)
        pltpu.sync_copy(rows_a, acc.at[dst_v.at[j]], add=True)

      plsc.subcore_barrier()

    def rbody(j, carry):
      pltpu.sync_copy(acc.at[pl.ds(base + j * RC, RC)],
                      rows_a.at[pl.ds(0, RC)])
      pltpu.sync_copy(rows_a.at[pl.ds(0, RC)],
                      out_hbm.at[cid, pl.ds(base + j * RC, RC)])
      pltpu.sync_copy(acc_deg.at[pl.ds(base + j * RC, RC)], zbuf)
      pltpu.sync_copy(zbuf, deg_hbm.at[cid, pl.ds(base + j * RC, RC)])
      return carry

    with jax.named_scope("p1_read"):
      lax.fori_loop(0, RCHUNKS, rbody, 0)

  return seg


def _sc_kernel_pass2():
  mesh = plsc.VectorSubcoreMesh(core_axis_name="c", subcore_axis_name="s")

  @functools.partial(
      pl.kernel,
      out_type=jax.ShapeDtypeStruct((2, NROWS, D2), jnp.float32),
      mesh=mesh,
      compiler_params=pltpu.CompilerParams(use_tc_tiling_on_sc=False),
      scratch_types=[
          pltpu.VMEM((IDXROWS, EC), jnp.int32),
          pltpu.VMEM((IDXROWS, EC), jnp.int32),
          pltpu.VMEM((EC, D2), jnp.float32),
          pltpu.VMEM((EC, D2), jnp.float32),
          pltpu.VMEM_SHARED((NROWS, D2), jnp.float32),
          pltpu.SemaphoreType.DMA,
          pltpu.SemaphoreType.DMA,
      ],
  )
  def seg(vals_hbm, src_hbm, dst_hbm, zd_hbm, out_hbm,
          src_v, dst_v, rows_a, rows_b, acc, sem_a, sem_b):
    cid = lax.axis_index("c")
    sid = lax.axis_index("s")
    stage, off, npairs, extra = _tile_plan(cid, sid)
    base = sid * ROWS_PER_TILE
    pltpu.sync_copy(zd_hbm, rows_a.at[pl.ds(0, RC)])

    def zbody(j, carry):
      pltpu.sync_copy(rows_a.at[pl.ds(0, RC)],
                      acc.at[pl.ds(base + j * RC, RC)])
      return carry

    lax.fori_loop(0, RCHUNKS, zbody, 0)
    pltpu.sync_copy(src_hbm.at[pl.ds(stage, IDXROWS)], src_v)
    pltpu.sync_copy(dst_hbm.at[pl.ds(stage, IDXROWS)], dst_v)
    plsc.subcore_barrier()

    def body(jj, carry):
      j0 = off + 2 * jj
      j1 = off + 2 * jj + 1
      ga = pltpu.async_copy(vals_hbm.at[src_v.at[j0]], rows_a, sem_a)
      gb = pltpu.async_copy(vals_hbm.at[src_v.at[j1]], rows_b, sem_b)
      ga.wait()
      pltpu.sync_copy(rows_a, acc.at[dst_v.at[j0]], add=True)
      gb.wait()
      pltpu.sync_copy(rows_b, acc.at[dst_v.at[j1]], add=True)
      return carry

    lax.fori_loop(0, npairs, body, 0)

    @pl.when(extra)
    def _():
      j = off + 2 * npairs
      pltpu.async_copy(vals_hbm.at[src_v.at[j]], rows_a, sem_a).wait()
      pltpu.sync_copy(rows_a, acc.at[dst_v.at[j]], add=True)

    plsc.subcore_barrier()

    def rbody(j, carry):
      pltpu.sync_copy(acc.at[pl.ds(base + j * RC, RC)],
                      rows_a.at[pl.ds(0, RC)])
      pltpu.sync_copy(rows_a.at[pl.ds(0, RC)],
                      out_hbm.at[cid, pl.ds(base + j * RC, RC)])
      return carry

    lax.fori_loop(0, RCHUNKS, rbody, 0)

  return seg


_sc_cache = {}


def _sc_pass1(*args):
  if 1 not in _sc_cache:
    _sc_cache[1] = _sc_kernel_pass1()
  return _sc_cache[1](*args)


def _sc_pass2(*args):
  if 2 not in _sc_cache:
    _sc_cache[2] = _sc_kernel_pass2()
  return _sc_cache[2](*args)


_BLK = 2000


def _dense1_body(part, degp, x, w1, w2, w3, b1, b3, wc2, b12, b32,
                 p_out, r_out):
  gx = part[0] + part[1]
  deg = degp[0, :, 0:1] + degp[1, :, 0:1]
  xd = (x[:].astype(jnp.float32) * (-deg)).astype(jnp.bfloat16)
  h = jnp.dot(gx, w1[:], preferred_element_type=jnp.float32)
  h = h + jnp.dot(xd, w2[:], preferred_element_type=jnp.float32)
  h = h + jnp.dot(x[:], w3[:], preferred_element_type=jnp.float32)
  h = h + deg * b1[:] + b3[:]
  h = jnp.maximum(h, 0.0)
  m2 = jnp.dot(h.astype(jnp.bfloat16), wc2[:],
               preferred_element_type=jnp.float32)
  p = m2[:, 0:OUT]
  p_out[:] = jnp.concatenate([p, jnp.zeros_like(p)], axis=1)
  r_out[:] = deg * b12[:] - deg * m2[:, OUT:2 * OUT] + m2[:, 2 * OUT:3 * OUT] \
      + b32[:]


def _dense2_body(gp, r, o):
  s = gp[0, :, 0:OUT] + gp[1, :, 0:OUT] + r[:]
  o[:] = jnp.maximum(s, 0.0)


def kernel(x, edge_index, W1_1, b1_1, W2_1, W3_1, b3_1,
           W1_2, b1_2, W2_2, W3_2, b3_2):
  src_p = edge_index[0].reshape(NCHUNKS, EC)
  dst_p = edge_index[1].reshape(NCHUNKS, EC)

  xb = x.astype(jnp.bfloat16)
  zeros1 = jnp.zeros((EC, IN), jnp.bfloat16)
  zerosd = jnp.zeros((RC, D2), jnp.float32)
  ones1 = jnp.ones((EC, D2), jnp.float32)
  part1, degp = _sc_pass1(xb, src_p, dst_p, zeros1, zerosd, ones1)

  wc2 = jnp.concatenate([W1_2, W2_2, W3_2],
                        axis=1).astype(jnp.bfloat16)
  grid = N // _BLK
  p_pad, r = pl.pallas_call(
      _dense1_body,
      grid=(grid,),
      in_specs=[
          pl.BlockSpec((2, _BLK, IN), lambda i: (0, i, 0)),
          pl.BlockSpec((2, _BLK, D2), lambda i: (0, i, 0)),
          pl.BlockSpec((_BLK, IN), lambda i: (i, 0)),
          pl.BlockSpec((IN, HID), lambda i: (0, 0)),
          pl.BlockSpec((IN, HID), lambda i: (0, 0)),
          pl.BlockSpec((IN, HID), lambda i: (0, 0)),
          pl.BlockSpec((1, HID), lambda i: (0, 0)),
          pl.BlockSpec((1, HID), lambda i: (0, 0)),
          pl.BlockSpec((HID, 3 * OUT), lambda i: (0, 0)),
          pl.BlockSpec((1, OUT), lambda i: (0, 0)),
          pl.BlockSpec((1, OUT), lambda i: (0, 0)),
      ],
      out_specs=[
          pl.BlockSpec((_BLK, D2), lambda i: (i, 0)),
          pl.BlockSpec((_BLK, OUT), lambda i: (i, 0)),
      ],
      out_shape=[
          jax.ShapeDtypeStruct((N, D2), jnp.float32),
          jax.ShapeDtypeStruct((N, OUT), jnp.float32),
      ],
  )(part1, degp, xb, W1_1.astype(jnp.bfloat16), W2_1.astype(jnp.bfloat16),
    W3_1.astype(jnp.bfloat16), b1_1.reshape(1, HID), b3_1.reshape(1, HID),
    wc2, b1_2.reshape(1, OUT), b3_2.reshape(1, OUT))

  part2 = _sc_pass2(p_pad, src_p, dst_p, zerosd)

  out = pl.pallas_call(
      _dense2_body,
      grid=(grid,),
      in_specs=[
          pl.BlockSpec((2, _BLK, D2), lambda i: (0, i, 0)),
          pl.BlockSpec((_BLK, OUT), lambda i: (i, 0)),
      ],
      out_specs=pl.BlockSpec((_BLK, OUT), lambda i: (i, 0)),
      out_shape=jax.ShapeDtypeStruct((N, OUT), jnp.float32),
  )(part2, r)
  return out
